# Initial kernel scaffold; baseline (speedup 1.0000x reference)
import jax
import jax.numpy as jnp
from jax.experimental import pallas as pl


def kernel(word_embs, cooc_idx, mask, wordmem, bankmem):
    raise NotImplementedError("write your pallas kernel here")


# speedup vs baseline: 6.1118x; 6.1118x over previous
import functools

import jax
import jax.numpy as jnp
from jax import lax
from jax.experimental import pallas as pl
from jax.experimental.pallas import tpu as pltpu
from jax.experimental.pallas import tpu_sc as plsc

N = 4096
L = 64
WD = 128
HD = 256
NW = 32
TPW = N // NW
LANES = 16
G = L // LANES


def _rsqrt(x):
    i = lax.bitcast_convert_type(x, jnp.int32)
    i = jnp.int32(0x5F3759DF) - (i >> 1)
    y = lax.bitcast_convert_type(i, jnp.float32)
    for _ in range(3):
        y = y * (1.5 - 0.5 * x * y * y)
    return y


def _splat_i32(v):
    return jnp.full((LANES,), v, jnp.int32)


_MESH = plsc.VectorSubcoreMesh(core_axis_name="c", subcore_axis_name="s")


@functools.partial(
    pl.kernel,
    mesh=_MESH,
    compiler_params=pltpu.CompilerParams(use_tc_tiling_on_sc=False,
                                          needs_layout_passes=False),
    out_type=jax.ShapeDtypeStruct((N, HD), jnp.float32),
    scratch_types=[
        pltpu.VMEM((TPW, WD), jnp.float32),
        pltpu.VMEM((TPW, L), jnp.int32),
        pltpu.VMEM((TPW, L), jnp.float32),
        pltpu.VMEM((L, WD), jnp.float32),
        pltpu.VMEM((L, HD), jnp.float32),
        pltpu.VMEM((L,), jnp.float32),
        pltpu.VMEM((TPW, HD), jnp.float32),
        pltpu.SemaphoreType.DMA,
        pltpu.SemaphoreType.DMA,
    ],
)
def _mb_kernel(q_hbm, idx_hbm, mask_hbm, wmem_hbm, bmem_hbm, out_hbm,
               q_v, idx_v, mask_v, wm_v, bm_v, att_v, out_v, sem0, sem1):
    wid = lax.axis_index("s") * 2 + lax.axis_index("c")
    base = wid * TPW
    pltpu.sync_copy(q_hbm.at[pl.ds(base, TPW)], q_v)
    pltpu.sync_copy(idx_hbm.at[pl.ds(base, TPW)], idx_v)
    pltpu.sync_copy(mask_hbm.at[pl.ds(base, TPW)], mask_v)

    lane = lax.iota(jnp.int32, LANES)

    def token_body(t, carry):
        idx_row = idx_v.at[t]
        cp0 = pltpu.async_copy(wmem_hbm.at[idx_row], wm_v, sem0)
        cp1 = pltpu.async_copy(bmem_hbm.at[idx_row], bm_v, sem1)
        cp0.wait()
        cp1.wait()

        def d_body(d, accs):
            qd = plsc.load_gather(q_v, [_splat_i32(t), _splat_i32(d)])
            dsp = _splat_i32(d)
            out = []
            for g in range(G):
                acc, nacc = accs[2 * g], accs[2 * g + 1]
                vals = plsc.load_gather(wm_v, [lane + (g * LANES), dsp])
                out.append(acc + vals * qd)
                out.append(nacc + vals * vals)
            return tuple(out)

        zero = jnp.zeros((LANES,), jnp.float32)
        accs = lax.fori_loop(0, WD, d_body, (zero,) * (2 * G))

        qn = zero
        for j in range(WD // LANES):
            v = q_v[t, pl.ds(j * LANES, LANES)]
            qn = qn + v * v
        qr = _rsqrt(jnp.maximum(jnp.full((LANES,), jnp.sum(qn)), 1e-24))

        es = []
        for g in range(G):
            acc, nacc = accs[2 * g], accs[2 * g + 1]
            s = acc * _rsqrt(jnp.maximum(nacc, 1e-24)) * qr
            es.append(jnp.exp(s) * mask_v[t, pl.ds(g * LANES, LANES)])
        den = jnp.full((LANES,), jnp.sum(es[0] + es[1] + es[2] + es[3]))
        for g in range(G):
            att_v[pl.ds(g * LANES, LANES)] = es[g] / den

        def l_body(l, outs):
            w = plsc.load_gather(att_v, [_splat_i32(l)])
            return tuple(outs[j] + w * bm_v[l, pl.ds(j * LANES, LANES)]
                         for j in range(HD // LANES))

        outs = lax.fori_loop(0, L, l_body, (zero,) * (HD // LANES))
        for j in range(HD // LANES):
            out_v[t, pl.ds(j * LANES, LANES)] = outs[j]
        return carry

    lax.fori_loop(0, TPW, token_body, 0)
    pltpu.sync_copy(out_v, out_hbm.at[pl.ds(base, TPW)])


def kernel(word_embs, cooc_idx, mask, wordmem, bankmem):
    return _mb_kernel(word_embs, cooc_idx.astype(jnp.int32), mask,
                      wordmem, bankmem)
